# baseline (speedup 1.0000x reference)
import functools

import jax
import jax.numpy as jnp
from jax import lax
from jax.experimental import pallas as pl
from jax.experimental.pallas import tpu as pltpu
from jax.experimental.pallas import tpu_sc as plsc

_B, _C, _N = 16, 512, 2048
_MAX_H, _MAX_W = 70, 40
_GRID = 300
_HW = _MAX_H * _MAX_W
_ZSLOT = _N
_NC, _NS = 2, 16
_CPT = _C // _NC


def _tc_pix_body(feat_ref, ys_ref, xs_ref, pix_ref):
    yb = ys_ref[0]
    xb = xs_ref[0]
    valid = yb > -1
    min_y = jnp.min(jnp.where(valid, yb, _GRID))
    max_y = jnp.max(jnp.where(valid, yb, -1))
    min_x = jnp.min(jnp.where(valid, xb, _GRID))
    max_x = jnp.max(jnp.where(valid, xb, -1))
    h0 = max_y - min_y + 1
    w0 = max_x - min_x + 1
    swap = w0 > h0
    height = jnp.where(swap, w0, h0)
    width = jnp.where(swap, h0, w0)
    h_dif = height - _MAX_H
    w_dif = width - _MAX_W
    cut_top = jnp.where(h_dif > 0, (h_dif + 1) // 2, 0)
    pad_top = jnp.where(h_dif > 0, 0, (-h_dif + 1) // 2)
    cut_left = jnp.where(w_dif > 0, (w_dif + 1) // 2, 0)
    pad_right = jnp.where(w_dif > 0, 0, (-w_dif + 1) // 2)
    ry = yb - min_y
    rx = xb - min_x
    row = jnp.where(swap, rx, ry)
    col = jnp.where(swap, ry, rx)
    r = row - cut_top + pad_top
    c = col - cut_left + pad_right
    inb = (r >= 0) & (r < _MAX_H) & (c >= 0) & (c < _MAX_W)
    f = feat_ref[0]
    real = jnp.min(jnp.where(f != -1.0, 1, 0).astype(jnp.int32),
                   axis=0, keepdims=True)
    pix = jnp.where(valid & inb & (real > 0), r * _MAX_W + c, -1)
    pix_ref[0] = pix.astype(jnp.int32)


def _tc_pix(features, ys3, xs3):
    return pl.pallas_call(
        _tc_pix_body,
        grid=(_B,),
        in_specs=[
            pl.BlockSpec((1, _C, _N), lambda b: (b, 0, 0)),
            pl.BlockSpec((1, 1, _N), lambda b: (b, 0, 0)),
            pl.BlockSpec((1, 1, _N), lambda b: (b, 0, 0)),
        ],
        out_specs=pl.BlockSpec((1, 1, _N), lambda b: (b, 0, 0)),
        out_shape=jax.ShapeDtypeStruct((_B, 1, _N), jnp.int32),
    )(features, ys3, xs3)


_G = 8
_NCHUNK = _CPT // _G
_TROW = _N + 16


@functools.lru_cache(maxsize=None)
def _sc_gather_fn():
    return functools.partial(
        pl.kernel,
        mesh=plsc.VectorSubcoreMesh(core_axis_name="c", subcore_axis_name="s"),
        compiler_params=pltpu.CompilerParams(
            needs_layout_passes=False, use_tc_tiling_on_sc=False),
        out_type=jax.ShapeDtypeStruct((_B * _C, _HW), jnp.float32),
        scratch_types=[
            pltpu.VMEM((_N,), jnp.int32),
            pltpu.VMEM((_HW,), jnp.int32),
            pltpu.VMEM((_G * _TROW,), jnp.float32),
            pltpu.VMEM((_G * _TROW,), jnp.float32),
            pltpu.VMEM((_G, _HW), jnp.float32),
            pltpu.VMEM((_G, _HW), jnp.float32),
            pltpu.SemaphoreType.DMA,
            pltpu.SemaphoreType.DMA,
            pltpu.SemaphoreType.DMA,
            pltpu.SemaphoreType.DMA,
        ],
    )(_sc_gather_body)


def _sc_gather_body(feat_hbm, pix_hbm, out_hbm, pix_v, imap_v,
                    tab0, tab1, ob0, ob1, sin0, sin1, sout0, sout1):
    cid = lax.axis_index("c")
    sid = lax.axis_index("s")
    b = sid
    c0 = cid * _CPT
    tabs, obs = (tab0, tab1), (ob0, ob1)
    sins, souts = (sin0, sin1), (sout0, sout1)

    def start_in(i, par):
        ch0 = c0 + i * _G
        for g in range(_G):
            pltpu.make_async_copy(
                feat_hbm.at[b, ch0 + g],
                tabs[par].at[pl.ds(g * _TROW, _N)], sins[par]).start()

    def wait_in(par):
        for g in range(_G):
            pltpu.make_async_copy(
                feat_hbm.at[b, c0 + g],
                tabs[par].at[pl.ds(g * _TROW, _N)], sins[par]).wait()

    def start_out(i, par):
        row0 = b * _C + c0 + i * _G
        pltpu.make_async_copy(obs[par], out_hbm.at[pl.ds(row0, _G)],
                              souts[par]).start()

    def wait_out(par):
        pltpu.make_async_copy(obs[par], out_hbm.at[pl.ds(b * _C, _G)],
                              souts[par]).wait()

    start_in(0, 0)
    start_in(1, 1)
    pltpu.sync_copy(pix_hbm.at[b], pix_v)
    z16 = jnp.zeros((16,), jnp.float32)
    for par in (0, 1):
        for g in range(_G):
            tabs[par][pl.ds(g * _TROW + _N, 16)] = z16
    zvec = jnp.full((16,), _ZSLOT, jnp.int32)
    for k in range(_HW // 16):
        imap_v[pl.ds(k * 16, 16)] = zvec
    iota16 = lax.iota(jnp.int32, 16)
    for j in range(_N // 16):
        idx = pix_v[pl.ds(j * 16, 16)]
        m = idx >= 0
        plsc.store_scatter(imap_v, [jnp.maximum(idx, 0)], iota16 + (j * 16), mask=m)

    def gather_plane(g, par):
        goff = g * _TROW
        for k in range(_HW // 16):
            im = imap_v[pl.ds(k * 16, 16)]
            obs[par][g, pl.ds(k * 16, 16)] = plsc.load_gather(
                tabs[par], [im + goff])
        return g

    def body2(j, carry):
        for par in (0, 1):
            i = 2 * j + par
            wait_in(par)

            @pl.when(i >= 2)
            def _():
                wait_out(par)

            lax.fori_loop(0, _G, lambda g, _: gather_plane(g, par), 0)
            start_out(i, par)

            @pl.when(i + 2 < _NCHUNK)
            def _():
                start_in(i + 2, par)
        return carry

    lax.fori_loop(0, _NCHUNK // 2, body2, 0)
    wait_out(0)
    wait_out(1)


def kernel(features, ys, xs):
    ys3 = ys.reshape(_B, 1, _N)
    xs3 = xs.reshape(_B, 1, _N)
    pix = _tc_pix(features, ys3, xs3)
    out = _sc_gather_fn()(features, pix.reshape(_B, _N))
    return out.reshape(_B, _C, _MAX_H, _MAX_W)


# speedup vs baseline: 2.2218x; 2.2218x over previous
import functools

import jax
import jax.numpy as jnp
from jax import lax
from jax.experimental import pallas as pl
from jax.experimental.pallas import tpu as pltpu
from jax.experimental.pallas import tpu_sc as plsc

_B, _C, _N = 16, 512, 2048
_MAX_H, _MAX_W = 70, 40
_GRID = 300
_HW = _MAX_H * _MAX_W
_HWP = 2816
_NT = _HWP // 128
_NC, _NS = 2, 16
_CPT = _C // _NC
_G = 8
_NCHUNK = _CPT // _G
_ZADDR = _G * _N


def _tc_pix_body(feat_ref, ys_ref, xs_ref, pix_ref):
    yb = ys_ref[0]
    xb = xs_ref[0]
    valid = yb > -1
    min_y = jnp.min(jnp.where(valid, yb, _GRID))
    max_y = jnp.max(jnp.where(valid, yb, -1))
    min_x = jnp.min(jnp.where(valid, xb, _GRID))
    max_x = jnp.max(jnp.where(valid, xb, -1))
    h0 = max_y - min_y + 1
    w0 = max_x - min_x + 1
    swap = w0 > h0
    height = jnp.where(swap, w0, h0)
    width = jnp.where(swap, h0, w0)
    h_dif = height - _MAX_H
    w_dif = width - _MAX_W
    cut_top = jnp.where(h_dif > 0, (h_dif + 1) // 2, 0)
    pad_top = jnp.where(h_dif > 0, 0, (-h_dif + 1) // 2)
    cut_left = jnp.where(w_dif > 0, (w_dif + 1) // 2, 0)
    pad_right = jnp.where(w_dif > 0, 0, (-w_dif + 1) // 2)
    ry = yb - min_y
    rx = xb - min_x
    row = jnp.where(swap, rx, ry)
    col = jnp.where(swap, ry, rx)
    r = row - cut_top + pad_top
    c = col - cut_left + pad_right
    inb = (r >= 0) & (r < _MAX_H) & (c >= 0) & (c < _MAX_W)
    f = feat_ref[0]
    real = jnp.min(jnp.where(f != -1.0, 1, 0).astype(jnp.int32),
                   axis=0, keepdims=True)
    pix = jnp.where(valid & inb & (real > 0), r * _MAX_W + c, -1)
    pix = pix.astype(jnp.int32)
    for t in range(_N // 128):
        pix_ref[0, pl.ds(t, 1), :] = pix[:, t * 128:(t + 1) * 128]


def _tc_pix(features, ys3, xs3):
    return pl.pallas_call(
        _tc_pix_body,
        grid=(_B,),
        in_specs=[
            pl.BlockSpec((1, _C, _N), lambda b: (b, 0, 0)),
            pl.BlockSpec((1, 1, _N), lambda b: (b, 0, 0)),
            pl.BlockSpec((1, 1, _N), lambda b: (b, 0, 0)),
        ],
        out_specs=pl.BlockSpec((1, _N // 128, 128), lambda b: (b, 0, 0)),
        out_shape=jax.ShapeDtypeStruct((_B, _N // 128, 128), jnp.int32),
    )(features, ys3, xs3)


@functools.lru_cache(maxsize=None)
def _sc_gather_fn():
    return functools.partial(
        pl.kernel,
        mesh=plsc.VectorSubcoreMesh(core_axis_name="c", subcore_axis_name="s"),
        compiler_params=pltpu.CompilerParams(needs_layout_passes=False),
        out_type=jax.ShapeDtypeStruct((_B * _C // _G * _NT, _G, 128),
                                      jnp.float32),
        scratch_types=[
            pltpu.VMEM((_N // 128, 128), jnp.int32),
            pltpu.VMEM((_HWP,), jnp.int32),
            pltpu.VMEM((2 * _G, _N), jnp.float32),
            pltpu.VMEM((2 * _G, _N), jnp.float32),
            pltpu.VMEM((_NT, _G, 128), jnp.float32),
            pltpu.VMEM((_NT, _G, 128), jnp.float32),
            pltpu.SemaphoreType.DMA,
            pltpu.SemaphoreType.DMA,
            pltpu.SemaphoreType.DMA,
            pltpu.SemaphoreType.DMA,
        ],
    )(_sc_gather_body)


def _sc_gather_body(feat_hbm, pix_hbm, out_hbm, pix_v, imap_v,
                    tab0, tab1, ob0, ob1, sin0, sin1, sout0, sout1):
    cid = lax.axis_index("c")
    sid = lax.axis_index("s")
    b = sid
    c0 = cid * _CPT
    tabs, obs = (tab0, tab1), (ob0, ob1)
    sins, souts = (sin0, sin1), (sout0, sout1)

    def start_in(i, par):
        pltpu.make_async_copy(
            feat_hbm.at[b, pl.ds(c0 + i * _G, _G)],
            tabs[par].at[pl.ds(0, _G)], sins[par]).start()

    def wait_in(par):
        pltpu.make_async_copy(
            feat_hbm.at[b, pl.ds(c0, _G)],
            tabs[par].at[pl.ds(0, _G)], sins[par]).wait()

    def start_out(i, par):
        q0 = (b * (_C // _G) + (c0 // _G) + i) * _NT
        pltpu.make_async_copy(obs[par], out_hbm.at[pl.ds(q0, _NT)],
                              souts[par]).start()

    def wait_out(par):
        pltpu.make_async_copy(obs[par], out_hbm.at[pl.ds(0, _NT)],
                              souts[par]).wait()

    start_in(0, 0)
    start_in(1, 1)
    pltpu.sync_copy(pix_hbm.at[b], pix_v)
    z16 = jnp.zeros((16,), jnp.float32)
    for par in (0, 1):
        for zr in range(_G, 2 * _G):
            for k in range(_N // 16):
                tabs[par][zr, pl.ds(k * 16, 16)] = z16
    zvec = jnp.full((16,), _ZADDR, jnp.int32)
    for k in range(_HWP // 16):
        imap_v[pl.ds(k * 16, 16)] = zvec
    iota16 = lax.iota(jnp.int32, 16)
    for j in range(_N // 16):
        idx = pix_v[j // 8, pl.ds((j % 8) * 16, 16)]
        m = idx >= 0
        v = iota16 + (j * 16)
        plsc.store_scatter(imap_v, [jnp.maximum(idx, 0)], v, mask=m)

    def gather_chunk(par):
        @plsc.parallel_loop(0, _NT, unroll=2)
        def _(t):
            for k8 in range(8):
                imt = imap_v[pl.ds(t * 128 + k8 * 16, 16)]
                for g in range(_G):
                    addr = imt + g * _N
                    v = plsc.load_gather(
                        tabs[par], [addr >> 11, addr & 2047])
                    obs[par][t, g, pl.ds(k8 * 16, 16)] = v

    def body2(j, carry):
        for par in (0, 1):
            i = 2 * j + par
            wait_in(par)

            @pl.when(i >= 2)
            def _():
                wait_out(par)

            gather_chunk(par)
            start_out(i, par)

            @pl.when(i + 2 < _NCHUNK)
            def _():
                start_in(i + 2, par)
        return carry

    lax.fori_loop(0, _NCHUNK // 2, body2, 0)
    wait_out(0)
    wait_out(1)


def kernel(features, ys, xs):
    ys3 = ys.reshape(_B, 1, _N)
    xs3 = xs.reshape(_B, 1, _N)
    pix = _tc_pix(features, ys3, xs3)
    out3 = _sc_gather_fn()(features, pix)
    out = (out3.reshape(_B * _C // _G, _NT, _G, 128)
           .transpose(0, 2, 1, 3)
           .reshape(_B * _C, _HWP)[:, :_HW])
    return out.reshape(_B, _C, _MAX_H, _MAX_W)
